# baseline (speedup 1.0000x reference)
import functools

import jax
import jax.numpy as jnp
from jax import lax
from jax.experimental import pallas as pl
from jax.experimental.pallas import tpu as pltpu
from jax.experimental.pallas import tpu_sc as plsc

N_NODES = 10000
N_EDGES = 160000
D_FEAT = 256

NC, NS, L = 2, 16, 16
NW = NC * NS
B = 128
N_CHUNKS = N_EDGES // B
BASE_CHUNKS = N_CHUNKS // NW
EXTRA = N_CHUNKS - BASE_CHUNKS * NW
GROUPS = B // L


def _rsqrt_newton(ss):
    ib = lax.bitcast_convert_type(ss, jnp.int32)
    seed = jnp.int32(0x5F3759DF) - lax.shift_right_logical(ib, 1)
    y = lax.bitcast_convert_type(seed, jnp.float32)
    for _ in range(3):
        y = y * (1.5 - 0.5 * ss * y * y)
    return y


_mesh = plsc.VectorSubcoreMesh(core_axis_name="c", subcore_axis_name="s")


@functools.partial(
    pl.kernel,
    mesh=_mesh,
    compiler_params=pltpu.CompilerParams(use_tc_tiling_on_sc=False,
                                         needs_layout_passes=False),
    out_type=jax.ShapeDtypeStruct((NW, L), jnp.float32),
    scratch_types=[
        pltpu.VMEM((B,), jnp.int32),
        pltpu.VMEM((B,), jnp.int32),
        pltpu.VMEM((B,), jnp.float32),
        pltpu.VMEM((B, D_FEAT), jnp.float32),
        pltpu.VMEM((B, D_FEAT), jnp.float32),
        pltpu.VMEM((L,), jnp.float32),
        pltpu.SemaphoreType.DMA,
        pltpu.SemaphoreType.DMA,
    ],
)
def _edge_partials(x_hbm, idx0_hbm, idx1_hbm, sd_hbm, out_hbm,
                   idx0_v, idx1_v, sd_v, rows0, rows1, accv, sem0, sem1):
    wid = lax.axis_index("s") * NC + lax.axis_index("c")
    n_chunks_w = BASE_CHUNKS + jnp.where(wid < EXTRA, 1, 0)
    lane = lax.iota(jnp.int32, L)

    def chunk_body(t, acc):
        c = wid + NW * t
        base = pl.multiple_of(c * B, B)
        pltpu.sync_copy(idx0_hbm.at[pl.ds(base, B)], idx0_v)
        pltpu.sync_copy(idx1_hbm.at[pl.ds(base, B)], idx1_v)
        pltpu.sync_copy(sd_hbm.at[pl.ds(base, B)], sd_v)
        cp0 = pltpu.async_copy(x_hbm.at[idx0_v], rows0, sem0)
        cp1 = pltpu.async_copy(x_hbm.at[idx1_v], rows1, sem1)
        cp0.wait()
        cp1.wait()
        for g in range(GROUPS):
            row_idx = lane + (g * L)

            def feat_body(d, ss):
                col = jnp.full((L,), 0, jnp.int32) + d
                a = plsc.load_gather(rows0, [row_idx, col])
                b = plsc.load_gather(rows1, [row_idx, col])
                df = a - b
                return ss + df * df

            ss = lax.fori_loop(0, D_FEAT, feat_body,
                               jnp.zeros((L,), jnp.float32))
            ss = jnp.maximum(ss, 1e-30)
            dist = ss * _rsqrt_newton(ss)
            r = sd_v[pl.ds(g * L, L)] - dist
            acc = acc + r * r
        return acc

    acc = lax.fori_loop(0, n_chunks_w, chunk_body,
                        jnp.zeros((L,), jnp.float32))
    accv[...] = acc
    pltpu.sync_copy(accv, out_hbm.at[wid])


def _sum_body(p_ref, o_ref):
    o_ref[0, 0] = jnp.sum(p_ref[...])


_sum_call = pl.pallas_call(
    _sum_body,
    out_shape=jax.ShapeDtypeStruct((1, 1), jnp.float32),
    out_specs=pl.BlockSpec(memory_space=pltpu.SMEM),
)


def kernel(input, edge_index, small_dists):
    ei = edge_index.astype(jnp.int32)
    idx0 = ei[:, 0]
    idx1 = ei[:, 1]
    partials = _edge_partials(input, idx0, idx1, small_dists)
    return _sum_call(partials)[0, 0]


# speedup vs baseline: 1.1902x; 1.1902x over previous
import functools

import jax
import jax.numpy as jnp
from jax import lax
from jax.experimental import pallas as pl
from jax.experimental.pallas import tpu as pltpu
from jax.experimental.pallas import tpu_sc as plsc

N_NODES = 10000
N_EDGES = 160000
D_FEAT = 256

NC, NS, L = 2, 16, 16
NW = NC * NS
EPW = N_EDGES // NW
B = 64
NFULL = EPW // B
TAIL = EPW - NFULL * B
GROUPS = B // L
UNROLL = 8


def _rsqrt_newton(ss):
    ib = lax.bitcast_convert_type(ss, jnp.int32)
    seed = jnp.int32(0x5F3759DF) - lax.shift_right_logical(ib, 1)
    y = lax.bitcast_convert_type(seed, jnp.float32)
    for _ in range(3):
        y = y * (1.5 - 0.5 * ss * y * y)
    return y


def _group_sumsq(rows0, rows1, row_idx):
    def feat_body(k, carry):
        s0, s1 = carry
        col0 = jnp.full((L,), k * UNROLL, jnp.int32)
        for dd in range(UNROLL):
            col = col0 + dd
            a = plsc.load_gather(rows0, [row_idx, col])
            b = plsc.load_gather(rows1, [row_idx, col])
            df = a - b
            if dd % 2 == 0:
                s0 = s0 + df * df
            else:
                s1 = s1 + df * df
        return s0, s1

    z = jnp.zeros((L,), jnp.float32)
    s0, s1 = lax.fori_loop(0, D_FEAT // UNROLL, feat_body, (z, z))
    return s0 + s1


def _edge_sqerr(ss, sd):
    ss = jnp.maximum(ss, 1e-30)
    dist = ss * _rsqrt_newton(ss)
    r = sd - dist
    return r * r


_mesh = plsc.VectorSubcoreMesh(core_axis_name="c", subcore_axis_name="s")


@functools.partial(
    pl.kernel,
    mesh=_mesh,
    compiler_params=pltpu.CompilerParams(use_tc_tiling_on_sc=False,
                                         needs_layout_passes=False),
    out_type=jax.ShapeDtypeStruct((NW, L), jnp.float32),
    scratch_types=[
        pltpu.VMEM((EPW + 16,), jnp.int32),
        pltpu.VMEM((EPW + 16,), jnp.int32),
        pltpu.VMEM((EPW + 16,), jnp.float32),
        pltpu.VMEM((B, D_FEAT), jnp.float32),
        pltpu.VMEM((B, D_FEAT), jnp.float32),
        pltpu.VMEM((B, D_FEAT), jnp.float32),
        pltpu.VMEM((B, D_FEAT), jnp.float32),
        pltpu.VMEM((L,), jnp.float32),
        pltpu.SemaphoreType.DMA,
        pltpu.SemaphoreType.DMA,
        pltpu.SemaphoreType.DMA,
        pltpu.SemaphoreType.DMA,
    ],
)
def _edge_partials(x_hbm, idx0_hbm, idx1_hbm, sd_hbm, out_hbm,
                   idx0_all, idx1_all, sd_all,
                   rows0a, rows1a, rows0b, rows1b, accv,
                   s0a, s1a, s0b, s1b):
    wid = lax.axis_index("s") * NC + lax.axis_index("c")
    e0 = pl.multiple_of(wid * EPW, 8)
    lane = lax.iota(jnp.int32, L)

    rows = ((rows0a, rows1a), (rows0b, rows1b))
    sems = ((s0a, s1a), (s0b, s1b))

    pltpu.sync_copy(idx0_hbm.at[pl.ds(e0, EPW)], idx0_all.at[pl.ds(0, EPW)])
    pltpu.sync_copy(idx1_hbm.at[pl.ds(e0, EPW)], idx1_all.at[pl.ds(0, EPW)])
    pltpu.sync_copy(sd_hbm.at[pl.ds(e0, EPW)], sd_all.at[pl.ds(0, EPW)])

    def issue(t, parity, n):
        r0, r1 = rows[parity]
        sm0, sm1 = sems[parity]
        pltpu.async_copy(x_hbm.at[idx0_all.at[pl.ds(t * B, n)]],
                         r0.at[pl.ds(0, n)], sm0)
        pltpu.async_copy(x_hbm.at[idx1_all.at[pl.ds(t * B, n)]],
                         r1.at[pl.ds(0, n)], sm1)

    def wait(t, parity, n):
        r0, r1 = rows[parity]
        sm0, sm1 = sems[parity]
        pltpu.make_async_copy(x_hbm.at[idx0_all.at[pl.ds(t * B, n)]],
                              r0.at[pl.ds(0, n)], sm0).wait()
        pltpu.make_async_copy(x_hbm.at[idx1_all.at[pl.ds(t * B, n)]],
                              r1.at[pl.ds(0, n)], sm1).wait()

    def compute(t, parity, acc):
        r0, r1 = rows[parity]
        for g in range(GROUPS):
            row_idx = lane + (g * L)
            ss = _group_sumsq(r0, r1, row_idx)
            sd = sd_all[pl.ds(t * B + g * L, L)]
            acc = acc + _edge_sqerr(ss, sd)
        return acc

    issue(0, 0, B)

    def outer_body(i, acc):
        tb = i * 2
        for par in range(2):
            t = tb + par

            @pl.when(t + 1 < NFULL)
            def _():
                issue(t + 1, 1 - par, B)

            wait(t, par, B)
            acc = compute(t, par, acc)
        return acc

    acc = lax.fori_loop(0, NFULL // 2, outer_body,
                        jnp.zeros((L,), jnp.float32))

    issue(NFULL, 0, TAIL)
    wait(NFULL, 0, TAIL)
    r0, r1 = rows[0]
    row_idx = jnp.bitwise_and(lane, TAIL - 1)
    ss = _group_sumsq(r0, r1, row_idx)
    sd = sd_all[pl.ds(NFULL * B, L)]
    sq = _edge_sqerr(ss, sd)
    acc = acc + jnp.where(lane < TAIL, sq, jnp.zeros((L,), jnp.float32))

    accv[...] = acc
    pltpu.sync_copy(accv, out_hbm.at[wid])


def _sum_body(p_ref, o_ref):
    o_ref[0, 0] = jnp.sum(p_ref[...])


_sum_call = pl.pallas_call(
    _sum_body,
    out_shape=jax.ShapeDtypeStruct((1, 1), jnp.float32),
    out_specs=pl.BlockSpec(memory_space=pltpu.SMEM),
)


def kernel(input, edge_index, small_dists):
    ei = edge_index.astype(jnp.int32)
    idx0 = ei[:, 0]
    idx1 = ei[:, 1]
    partials = _edge_partials(input, idx0, idx1, small_dists)
    return _sum_call(partials)[0, 0]


# speedup vs baseline: 8.3935x; 7.0519x over previous
import functools

import jax
import jax.numpy as jnp
from jax import lax
from jax.experimental import pallas as pl
from jax.experimental.pallas import tpu as pltpu
from jax.experimental.pallas import tpu_sc as plsc

N_NODES = 10000
N_EDGES = 160000
D_FEAT = 256

NC, NS, L = 2, 16, 16
NW = NC * NS
EPW = N_EDGES // NW
B = 64
NFULL = EPW // B
TAIL = EPW - NFULL * B
GROUPS = B // L
UNROLL = 8


def _rsqrt_newton(ss):
    ib = lax.bitcast_convert_type(ss, jnp.int32)
    seed = jnp.int32(0x5F3759DF) - lax.shift_right_logical(ib, 1)
    y = lax.bitcast_convert_type(seed, jnp.float32)
    for _ in range(3):
        y = y * (1.5 - 0.5 * ss * y * y)
    return y


def _group_sumsq(rows0, rows1, row_idx, lane):
    def feat_body(k, carry):
        s0, s1 = carry
        col0 = lane + (k * UNROLL)
        for dd in range(UNROLL):
            col = jnp.bitwise_and(col0 + dd, D_FEAT - 1)
            a = plsc.load_gather(rows0, [row_idx, col])
            b = plsc.load_gather(rows1, [row_idx, col])
            df = a - b
            if dd % 2 == 0:
                s0 = s0 + df * df
            else:
                s1 = s1 + df * df
        return s0, s1

    z = jnp.zeros((L,), jnp.float32)
    s0, s1 = lax.fori_loop(0, D_FEAT // UNROLL, feat_body, (z, z))
    return s0 + s1


def _edge_sqerr(ss, sd):
    ss = jnp.maximum(ss, 1e-30)
    dist = ss * _rsqrt_newton(ss)
    r = sd - dist
    return r * r


_mesh = plsc.VectorSubcoreMesh(core_axis_name="c", subcore_axis_name="s")


@functools.partial(
    pl.kernel,
    mesh=_mesh,
    compiler_params=pltpu.CompilerParams(use_tc_tiling_on_sc=False,
                                         needs_layout_passes=False),
    out_type=jax.ShapeDtypeStruct((NW, L), jnp.float32),
    scratch_types=[
        pltpu.VMEM((EPW + 16,), jnp.int32),
        pltpu.VMEM((EPW + 16,), jnp.int32),
        pltpu.VMEM((EPW + 16,), jnp.float32),
        pltpu.VMEM((B, D_FEAT), jnp.float32),
        pltpu.VMEM((B, D_FEAT), jnp.float32),
        pltpu.VMEM((B, D_FEAT), jnp.float32),
        pltpu.VMEM((B, D_FEAT), jnp.float32),
        pltpu.VMEM((L,), jnp.float32),
        pltpu.SemaphoreType.DMA,
        pltpu.SemaphoreType.DMA,
        pltpu.SemaphoreType.DMA,
        pltpu.SemaphoreType.DMA,
    ],
)
def _edge_partials(x_hbm, idx0_hbm, idx1_hbm, sd_hbm, out_hbm,
                   idx0_all, idx1_all, sd_all,
                   rows0a, rows1a, rows0b, rows1b, accv,
                   s0a, s1a, s0b, s1b):
    wid = lax.axis_index("s") * NC + lax.axis_index("c")
    e0 = pl.multiple_of(wid * EPW, 8)
    lane = lax.iota(jnp.int32, L)

    rows = ((rows0a, rows1a), (rows0b, rows1b))
    sems = ((s0a, s1a), (s0b, s1b))

    pltpu.sync_copy(idx0_hbm.at[pl.ds(e0, EPW)], idx0_all.at[pl.ds(0, EPW)])
    pltpu.sync_copy(idx1_hbm.at[pl.ds(e0, EPW)], idx1_all.at[pl.ds(0, EPW)])
    pltpu.sync_copy(sd_hbm.at[pl.ds(e0, EPW)], sd_all.at[pl.ds(0, EPW)])

    def issue(t, parity, n):
        r0, r1 = rows[parity]
        sm0, sm1 = sems[parity]
        pltpu.async_copy(x_hbm.at[idx0_all.at[pl.ds(t * B, n)]],
                         r0.at[pl.ds(0, n)], sm0)
        pltpu.async_copy(x_hbm.at[idx1_all.at[pl.ds(t * B, n)]],
                         r1.at[pl.ds(0, n)], sm1)

    def wait(t, parity, n):
        r0, r1 = rows[parity]
        sm0, sm1 = sems[parity]
        pltpu.make_async_copy(x_hbm.at[idx0_all.at[pl.ds(t * B, n)]],
                              r0.at[pl.ds(0, n)], sm0).wait()
        pltpu.make_async_copy(x_hbm.at[idx1_all.at[pl.ds(t * B, n)]],
                              r1.at[pl.ds(0, n)], sm1).wait()

    def compute(t, parity, acc):
        r0, r1 = rows[parity]
        for g in range(GROUPS):
            row_idx = lane + (g * L)
            ss = _group_sumsq(r0, r1, row_idx, lane)
            sd = sd_all[pl.ds(t * B + g * L, L)]
            acc = acc + _edge_sqerr(ss, sd)
        return acc

    issue(0, 0, B)

    def outer_body(i, acc):
        tb = i * 2
        for par in range(2):
            t = tb + par

            @pl.when(t + 1 < NFULL)
            def _():
                issue(t + 1, 1 - par, B)

            wait(t, par, B)
            acc = compute(t, par, acc)
        return acc

    acc = lax.fori_loop(0, NFULL // 2, outer_body,
                        jnp.zeros((L,), jnp.float32))

    issue(NFULL, 0, TAIL)
    wait(NFULL, 0, TAIL)
    r0, r1 = rows[0]
    row_idx = jnp.bitwise_and(lane, TAIL - 1)
    ss = _group_sumsq(r0, r1, row_idx, lane)
    sd = sd_all[pl.ds(NFULL * B, L)]
    sq = _edge_sqerr(ss, sd)
    acc = acc + jnp.where(lane < TAIL, sq, jnp.zeros((L,), jnp.float32))

    accv[...] = acc
    pltpu.sync_copy(accv, out_hbm.at[wid])


def _sum_body(p_ref, o_ref):
    o_ref[0, 0] = jnp.sum(p_ref[...])


_sum_call = pl.pallas_call(
    _sum_body,
    out_shape=jax.ShapeDtypeStruct((1, 1), jnp.float32),
    out_specs=pl.BlockSpec(memory_space=pltpu.SMEM),
)


def kernel(input, edge_index, small_dists):
    ei = edge_index.astype(jnp.int32)
    idx0 = ei[:, 0]
    idx1 = ei[:, 1]
    partials = _edge_partials(input, idx0, idx1, small_dists)
    return _sum_call(partials)[0, 0]
